# baseline (speedup 1.0000x reference)
import functools

import jax
import jax.numpy as jnp
from jax import lax
from jax.experimental import pallas as pl
from jax.experimental.pallas import tpu as pltpu
from jax.experimental.pallas import tpu_sc as plsc

WORD_VOCAB = 100000
TOPIC_VOCAB = 1000
ACT_VOCAB = 1000
WORD_D = 64
TOPIC_D = 32
ACT_D = 32
B = 4096
L = 200

_NC = 2
_NS = 16
_NW = _NC * _NS

_B_PER = B // _NW
_G1 = 104
_G2 = 104
_G2_OFF = L - _G2


def _body(word_ids, topic_ids, act_ids, word_tab, topic_tab, act_tab,
          word_out, topic_out, act_out,
          idx_all, buf_a, buf_b, tidx_v, trows_v, aidx_v, arows_v,
          gsem, osem, tsem):
  c = lax.axis_index("c")
  s = lax.axis_index("s")
  wid = s * _NC + c

  sb = wid * _B_PER
  pltpu.sync_copy(topic_ids.at[pl.ds(sb, _B_PER)], tidx_v)
  pltpu.sync_copy(act_ids.at[pl.ds(sb, _B_PER)], aidx_v)
  dt = pltpu.async_copy(topic_tab.at[tidx_v], trows_v, tsem)
  da = pltpu.async_copy(act_tab.at[aidx_v], arows_v, tsem)

  pltpu.sync_copy(word_ids.at[pl.ds(sb, _B_PER)], idx_all)

  def fire(bi, buf):
    return [
        pltpu.async_copy(word_tab.at[idx_all.at[bi, pl.ds(0, _G1)]],
                         buf.at[pl.ds(0, _G1)], gsem),
        pltpu.async_copy(word_tab.at[idx_all.at[bi, pl.ds(_G2_OFF, _G2)]],
                         buf.at[pl.ds(_G2_OFF, _G2)], gsem),
    ]

  def out_start(bi, buf):
    pltpu.async_copy(buf, word_out.at[sb + bi], osem)

  def out_wait():
    pltpu.make_async_copy(buf_a, word_out.at[sb], osem).wait()

  for bi, buf in ((0, buf_a), (1, buf_b)):
    for d in fire(bi, buf):
      d.wait()
    out_start(bi, buf)

  @pl.loop(1, _B_PER // 2)
  def _w(ci):
    for b, buf in ((0, buf_a), (1, buf_b)):
      bi = ci * 2 + b
      out_wait()
      for d in fire(bi, buf):
        d.wait()
      out_start(bi, buf)

  out_wait()
  out_wait()

  dt.wait()
  da.wait()
  pltpu.sync_copy(trows_v, topic_out.at[pl.ds(sb, _B_PER)])
  pltpu.sync_copy(arows_v, act_out.at[pl.ds(sb, _B_PER)])


@jax.jit
def _run(word_ids, topic_ids, act_ids, word_table, topic_table, act_table):
  mesh = plsc.VectorSubcoreMesh(core_axis_name="c", subcore_axis_name="s")
  k = pl.kernel(
      _body,
      out_type=(
          jax.ShapeDtypeStruct((B, L, WORD_D), jnp.float32),
          jax.ShapeDtypeStruct((B, TOPIC_D), jnp.float32),
          jax.ShapeDtypeStruct((B, ACT_D), jnp.float32),
      ),
      mesh=mesh,
      scratch_types=(
          pltpu.VMEM((_B_PER, L), jnp.int32),
          pltpu.VMEM((L, WORD_D), jnp.float32),
          pltpu.VMEM((L, WORD_D), jnp.float32),
          pltpu.VMEM((_B_PER,), jnp.int32),
          pltpu.VMEM((_B_PER, TOPIC_D), jnp.float32),
          pltpu.VMEM((_B_PER,), jnp.int32),
          pltpu.VMEM((_B_PER, ACT_D), jnp.float32),
          pltpu.SemaphoreType.DMA,
          pltpu.SemaphoreType.DMA,
          pltpu.SemaphoreType.DMA,
      ),
      compiler_params=pltpu.CompilerParams(use_tc_tiling_on_sc=False),
  )
  return k(word_ids, topic_ids, act_ids, word_table, topic_table, act_table)


def kernel(word_ids, topic_ids, act_ids, word_table, topic_table, act_table):
  wout, tout, aout = _run(word_ids.astype(jnp.int32),
                          topic_ids.astype(jnp.int32),
                          act_ids.astype(jnp.int32),
                          word_table, topic_table, act_table)
  return (wout, tout, aout)


# speedup vs baseline: 1.1338x; 1.1338x over previous
import functools

import jax
import jax.numpy as jnp
from jax import lax
from jax.experimental import pallas as pl
from jax.experimental.pallas import tpu as pltpu
from jax.experimental.pallas import tpu_sc as plsc

WORD_VOCAB = 100000
TOPIC_VOCAB = 1000
ACT_VOCAB = 1000
WORD_D = 64
TOPIC_D = 32
ACT_D = 32
B = 4096
L = 200

_NC = 2
_NS = 16
_NW = _NC * _NS

_TOTAL_W = B * L
_W_PER = _TOTAL_W // _NW
_SUPER = 128
_NSUPER = _W_PER // _SUPER
_S_PER = B // _NW
_LANES = 16


def _body(word_ids, topic_ids, act_ids, over_w, over_t, over_a,
          word_out, topic_out, act_out,
          g_a, g_b, t_a, t_b, i_a, i_b, tidx_v, aidx_v, tt_v, at_v,
          gsem_a, gsem_b, osem_a, osem_b, isem_a, isem_b, tsem):
  c = lax.axis_index("c")
  s = lax.axis_index("s")
  wid = s * _NC + c

  idx_base = wid * _W_PER
  out_base = wid * _W_PER

  def iload(i, ib, isem):
    pltpu.async_copy(word_ids.at[pl.ds(idx_base + i * _SUPER, _SUPER)],
                     ib, isem)

  def iwait(ib, isem):
    pltpu.make_async_copy(word_ids.at[pl.ds(0, _SUPER)], ib, isem).wait()

  def fire(gb, ib, gs):
    pltpu.async_copy(over_w.at[ib], gb, gs)

  def gwait(gb, ib, gs):
    pltpu.make_async_copy(over_w.at[ib], gb, gs).wait()

  def compact(gb, tb, width):
    nregs = width // _LANES
    @pl.loop(0, _SUPER, unroll=8)
    def _cp(r):
      for cc in range(nregs):
        tb[r, pl.ds(cc * _LANES, _LANES)] = gb[r, pl.ds(cc * _LANES, _LANES)]

  def out_start(i, tb, os):
    pltpu.async_copy(tb, word_out.at[pl.ds(out_base + i * _SUPER, _SUPER)], os)

  def out_wait(tb, os):
    pltpu.make_async_copy(tb, word_out.at[pl.ds(out_base, _SUPER)], os).wait()

  slots = ((g_a, t_a, i_a, gsem_a, osem_a, isem_a),
           (g_b, t_b, i_b, gsem_b, osem_b, isem_b))

  for i, (gb, tb, ib, gs, os, isem) in ((0, slots[0]), (1, slots[1])):
    iload(i, ib, isem)
  for i, (gb, tb, ib, gs, os, isem) in ((0, slots[0]), (1, slots[1])):
    iwait(ib, isem)
    fire(gb, ib, gs)
  for i, (gb, tb, ib, gs, os, isem) in ((0, slots[0]), (1, slots[1])):
    gwait(gb, ib, gs)
    iload(i + 2, ib, isem)
    compact(gb, tb, WORD_D)
    iwait(ib, isem)
    fire(gb, ib, gs)
    out_start(i, tb, os)

  @pl.loop(1, _NSUPER // 2 - 1)
  def _w(ci):
    for slot, (gb, tb, ib, gs, os, isem) in enumerate(slots):
      i = ci * 2 + slot
      gwait(gb, ib, gs)
      iload(i + 2, ib, isem)
      out_wait(tb, os)
      compact(gb, tb, WORD_D)
      iwait(ib, isem)
      fire(gb, ib, gs)
      out_start(i, tb, os)

  for i, (gb, tb, ib, gs, os, isem) in ((_NSUPER - 2, slots[0]),
                                        (_NSUPER - 1, slots[1])):
    gwait(gb, ib, gs)
    out_wait(tb, os)
    compact(gb, tb, WORD_D)
    out_start(i, tb, os)
  for gb, tb, ib, gs, os, isem in slots:
    out_wait(tb, os)

  sb = wid * _S_PER
  pltpu.sync_copy(topic_ids.at[pl.ds(sb, _S_PER)], tidx_v)
  pltpu.sync_copy(act_ids.at[pl.ds(sb, _S_PER)], aidx_v)
  dt = pltpu.async_copy(over_t.at[tidx_v], g_a, tsem)
  da = pltpu.async_copy(over_a.at[aidx_v], g_b, tsem)
  dt.wait()
  da.wait()
  compact(g_a, tt_v, TOPIC_D)
  compact(g_b, at_v, ACT_D)
  pltpu.sync_copy(tt_v, topic_out.at[pl.ds(sb, _S_PER)])
  pltpu.sync_copy(at_v, act_out.at[pl.ds(sb, _S_PER)])


@jax.jit
def _run(word_ids1d, topic_ids, act_ids, over_w, over_t, over_a):
  mesh = plsc.VectorSubcoreMesh(core_axis_name="c", subcore_axis_name="s")
  k = pl.kernel(
      _body,
      out_type=(
          jax.ShapeDtypeStruct((_TOTAL_W, WORD_D), jnp.float32),
          jax.ShapeDtypeStruct((B, TOPIC_D), jnp.float32),
          jax.ShapeDtypeStruct((B, ACT_D), jnp.float32),
      ),
      mesh=mesh,
      scratch_types=(
          pltpu.VMEM((_SUPER, 128), jnp.float32),
          pltpu.VMEM((_SUPER, 128), jnp.float32),
          pltpu.VMEM((_SUPER, WORD_D), jnp.float32),
          pltpu.VMEM((_SUPER, WORD_D), jnp.float32),
          pltpu.VMEM((_SUPER,), jnp.int32),
          pltpu.VMEM((_SUPER,), jnp.int32),
          pltpu.VMEM((_S_PER,), jnp.int32),
          pltpu.VMEM((_S_PER,), jnp.int32),
          pltpu.VMEM((_S_PER, TOPIC_D), jnp.float32),
          pltpu.VMEM((_S_PER, ACT_D), jnp.float32),
          pltpu.SemaphoreType.DMA,
          pltpu.SemaphoreType.DMA,
          pltpu.SemaphoreType.DMA,
          pltpu.SemaphoreType.DMA,
          pltpu.SemaphoreType.DMA,
          pltpu.SemaphoreType.DMA,
          pltpu.SemaphoreType.DMA,
      ),
  )
  return k(word_ids1d, topic_ids, act_ids, over_w, over_t, over_a)


def _overlap(table, d):
  n_shift = 128 // d
  parts = [table]
  shifted = table
  zrow = jnp.zeros((1, d), table.dtype)
  for _ in range(n_shift - 1):
    shifted = jnp.concatenate([shifted[1:], zrow], axis=0)
    parts.append(shifted)
  return jnp.concatenate(parts, axis=1)


def kernel(word_ids, topic_ids, act_ids, word_table, topic_table, act_table):
  word_ids1d = word_ids.reshape(_TOTAL_W).astype(jnp.int32)
  wout, tout, aout = _run(word_ids1d,
                          topic_ids.astype(jnp.int32),
                          act_ids.astype(jnp.int32),
                          _overlap(word_table, WORD_D),
                          _overlap(topic_table, TOPIC_D),
                          _overlap(act_table, ACT_D))
  return (wout.reshape(B, L, WORD_D), tout, aout)
